# Initial kernel scaffold; baseline (speedup 1.0000x reference)
import jax
import jax.numpy as jnp
from jax.experimental import pallas as pl


def kernel(x, rope, Wq, bq, Wk, bk, Wv, bv, gq, gk, Wo, bo):
    raise NotImplementedError("write your pallas kernel here")


# speedup vs baseline: 1.1467x; 1.1467x over previous
import functools
import math

import jax
import jax.numpy as jnp
from jax.experimental import pallas as pl
from jax.experimental.pallas import tpu as pltpu

S = 2048
C = 768
HD = 64
H = C // HD
EPS = float(jnp.finfo(jnp.float32).eps)

BS = 512
BQ = 512


def _qkv_kernel(x_ref, wqt_ref, wkt_ref, wvt_ref, b_ref,
                cgq_ref, sgq_ref, cgk_ref, sgk_ref, e_ref,
                q_ref, k_ref, v_ref):
    xb = x_ref[:]
    e = e_ref[:]

    q = jnp.dot(xb, wqt_ref[:], preferred_element_type=jnp.float32) + b_ref[0:1, :]
    k = jnp.dot(xb, wkt_ref[:], preferred_element_type=jnp.float32) + b_ref[1:2, :]
    v = jnp.dot(xb, wvt_ref[:], preferred_element_type=jnp.float32) + b_ref[2:3, :]

    def headnorm(t):
        ss = jnp.dot(t * t, e, preferred_element_type=jnp.float32,
                     precision=jax.lax.Precision.HIGHEST)
        ssb = jax.lax.dot_general(ss, e, (((1,), (1,)), ((), ())),
                                  preferred_element_type=jnp.float32,
                                  precision=jax.lax.Precision.HIGHEST)
        return t * jax.lax.rsqrt(ssb * (1.0 / HD) + EPS)

    lane = jax.lax.broadcasted_iota(jnp.int32, (BS, C), 1)
    even = (lane % 2) == 0

    def rope(t, cg, sg):
        left = pltpu.roll(t, C - 1, 1)
        right = pltpu.roll(t, 1, 1)
        partner = jnp.where(even, -left, right)
        return t * cg + partner * sg

    qr = rope(headnorm(q), cgq_ref[:], sgq_ref[:])
    kr = rope(headnorm(k), cgk_ref[:], sgk_ref[:])

    for h in range(H):
        sl = slice(h * HD, (h + 1) * HD)
        q_ref[h] = qr[:, sl]
        k_ref[h] = kr[:, sl]
        v_ref[h] = v[:, sl]


def _attn_kernel(q_ref, k_ref, v_ref, wo_ref, bo_ref, o_ref):
    h = pl.program_id(1)
    qb = q_ref[0]
    kb = k_ref[0]
    vb = v_ref[0]
    s = jax.lax.dot_general(qb, kb, (((1,), (1,)), ((), ())),
                            preferred_element_type=jnp.float32)
    s = s * (1.0 / math.sqrt(HD))
    m = jnp.max(s, axis=1, keepdims=True)
    p = jnp.exp(s - m)
    denom = jnp.sum(p, axis=1, keepdims=True)
    outh = jnp.dot(p, vb, preferred_element_type=jnp.float32) * (1.0 / denom)
    contrib = jnp.dot(outh, wo_ref[:], preferred_element_type=jnp.float32)

    @pl.when(h == 0)
    def _():
        o_ref[:] = contrib + bo_ref[:]

    @pl.when(h != 0)
    def _():
        o_ref[:] += contrib


@jax.jit
def kernel(x, rope, Wq, bq, Wk, bk, Wv, bv, gq, gk, Wo, bo):
    f32 = jnp.float32
    cos = rope[:, :, 0]
    sin = rope[:, :, 1]
    cos_e = jnp.repeat(cos, 2, axis=1)
    sin_e = jnp.repeat(sin, 2, axis=1)
    cos_full = jnp.tile(cos_e, (1, H))
    sin_full = jnp.tile(sin_e, (1, H))

    def pairswap(v):
        v2 = v.reshape(-1, 2)
        return jnp.stack([v2[:, 1], v2[:, 0]], axis=-1).reshape(-1)

    gq_t = jnp.tile(gq, H)
    gk_t = jnp.tile(gk, H)
    cgq = cos_full * gq_t[None, :]
    sgq = sin_full * pairswap(gq_t)[None, :]
    cgk = cos_full * gk_t[None, :]
    sgk = sin_full * pairswap(gk_t)[None, :]

    b_all = jnp.stack([bq, bk, bv])
    eye = jnp.repeat(jnp.eye(H, dtype=f32), HD, axis=0)

    row_spec = pl.BlockSpec((BS, C), lambda i: (i, 0))
    full_spec = pl.BlockSpec((C, C), lambda i: (0, 0))
    qkv_out_spec = pl.BlockSpec((H, BS, HD), lambda i: (0, i, 0))

    q3, k3, v3 = pl.pallas_call(
        _qkv_kernel,
        grid=(S // BS,),
        in_specs=[
            row_spec,
            full_spec, full_spec, full_spec,
            pl.BlockSpec((3, C), lambda i: (0, 0)),
            row_spec, row_spec, row_spec, row_spec,
            pl.BlockSpec((C, H), lambda i: (0, 0)),
        ],
        out_specs=[qkv_out_spec, qkv_out_spec, qkv_out_spec],
        out_shape=[jax.ShapeDtypeStruct((H, S, HD), f32)] * 3,
    )(x, Wq.T, Wk.T, Wv.T, b_all, cgq, sgq, cgk, sgk, eye)

    out = pl.pallas_call(
        _attn_kernel,
        grid=(S // BQ, H),
        in_specs=[
            pl.BlockSpec((1, BQ, HD), lambda i, h: (h, i, 0)),
            pl.BlockSpec((1, S, HD), lambda i, h: (h, 0, 0)),
            pl.BlockSpec((1, S, HD), lambda i, h: (h, 0, 0)),
            pl.BlockSpec((HD, C), lambda i, h: (h, 0)),
            pl.BlockSpec((1, C), lambda i, h: (0, 0)),
        ],
        out_specs=pl.BlockSpec((BQ, C), lambda i, h: (i, 0)),
        out_shape=jax.ShapeDtypeStruct((S, C), f32),
    )(q3, k3, v3, Wo.T, bo[None, :])

    return out


# speedup vs baseline: 1.2863x; 1.1217x over previous
import functools
import math

import jax
import jax.numpy as jnp
from jax.experimental import pallas as pl
from jax.experimental.pallas import tpu as pltpu

S = 2048
C = 768
HD = 64
H = C // HD
EPS = float(jnp.finfo(jnp.float32).eps)

BS = 512
BQ = 1024


def _qkv_kernel(x_ref, wqt_ref, wkt_ref, wvt_ref, b_ref,
                cgq_ref, sgq_ref, cgk_ref, sgk_ref, e_ref,
                q_ref, k_ref, v_ref):
    xb = x_ref[:]
    e = e_ref[:]

    q = jnp.dot(xb, wqt_ref[:], preferred_element_type=jnp.float32) + b_ref[0:1, :]
    k = jnp.dot(xb, wkt_ref[:], preferred_element_type=jnp.float32) + b_ref[1:2, :]
    v = jnp.dot(xb, wvt_ref[:], preferred_element_type=jnp.float32) + b_ref[2:3, :]

    def split_dot(t, dims):
        hi = t.astype(jnp.bfloat16).astype(jnp.float32)
        lo = t - hi
        f = lambda a: jax.lax.dot_general(a, e, dims,
                                          preferred_element_type=jnp.float32)
        return f(hi) + f(lo)

    def headnorm(t):
        ss = split_dot(t * t, (((1,), (0,)), ((), ())))
        r = jax.lax.rsqrt(ss * (1.0 / HD) + EPS)
        rb = split_dot(r, (((1,), (1,)), ((), ())))
        return t * rb

    lane = jax.lax.broadcasted_iota(jnp.int32, (BS, C), 1)
    even = (lane % 2) == 0

    def rope(t, cg, sg):
        left = pltpu.roll(t, C - 1, 1)
        right = pltpu.roll(t, 1, 1)
        partner = jnp.where(even, -left, right)
        return t * cg + partner * sg

    qr = rope(headnorm(q), cgq_ref[:], sgq_ref[:])
    kr = rope(headnorm(k), cgk_ref[:], sgk_ref[:])

    for h in range(H):
        sl = slice(h * HD, (h + 1) * HD)
        q_ref[h] = qr[:, sl]
        k_ref[h] = kr[:, sl]
        v_ref[h] = v[:, sl]


def _attn_kernel(q_ref, k_ref, v_ref, wo_ref, bo_ref, o_ref):
    h = pl.program_id(1)
    qb = q_ref[0]
    kb = k_ref[0]
    vb = v_ref[0]
    s = jax.lax.dot_general(qb, kb, (((1,), (1,)), ((), ())),
                            preferred_element_type=jnp.float32)
    s = s * (1.0 / math.sqrt(HD))
    m = jnp.max(s, axis=1, keepdims=True)
    p = jnp.exp(s - m)
    denom = jnp.sum(p, axis=1, keepdims=True)
    outh = jnp.dot(p, vb, preferred_element_type=jnp.float32) * (1.0 / denom)
    contrib = jnp.dot(outh, wo_ref[:], preferred_element_type=jnp.float32)

    @pl.when(h == 0)
    def _():
        o_ref[:] = contrib + bo_ref[:]

    @pl.when(h != 0)
    def _():
        o_ref[:] += contrib


@jax.jit
def kernel(x, rope, Wq, bq, Wk, bk, Wv, bv, gq, gk, Wo, bo):
    f32 = jnp.float32
    cos = rope[:, :, 0]
    sin = rope[:, :, 1]
    cos_e = jnp.repeat(cos, 2, axis=1)
    sin_e = jnp.repeat(sin, 2, axis=1)
    cos_full = jnp.tile(cos_e, (1, H))
    sin_full = jnp.tile(sin_e, (1, H))

    def pairswap(v):
        v2 = v.reshape(-1, 2)
        return jnp.stack([v2[:, 1], v2[:, 0]], axis=-1).reshape(-1)

    gq_t = jnp.tile(gq, H)
    gk_t = jnp.tile(gk, H)
    cgq = cos_full * gq_t[None, :]
    sgq = sin_full * pairswap(gq_t)[None, :]
    cgk = cos_full * gk_t[None, :]
    sgk = sin_full * pairswap(gk_t)[None, :]

    b_all = jnp.stack([bq, bk, bv])
    eye = jnp.repeat(jnp.eye(H, dtype=f32), HD, axis=0)

    row_spec = pl.BlockSpec((BS, C), lambda i: (i, 0))
    full_spec = pl.BlockSpec((C, C), lambda i: (0, 0))
    qkv_out_spec = pl.BlockSpec((H, BS, HD), lambda i: (0, i, 0))

    q3, k3, v3 = pl.pallas_call(
        _qkv_kernel,
        grid=(S // BS,),
        in_specs=[
            row_spec,
            full_spec, full_spec, full_spec,
            pl.BlockSpec((3, C), lambda i: (0, 0)),
            row_spec, row_spec, row_spec, row_spec,
            pl.BlockSpec((C, H), lambda i: (0, 0)),
        ],
        out_specs=[qkv_out_spec, qkv_out_spec, qkv_out_spec],
        out_shape=[jax.ShapeDtypeStruct((H, S, HD), f32)] * 3,
    )(x, Wq.T, Wk.T, Wv.T, b_all, cgq, sgq, cgk, sgk, eye)

    out = pl.pallas_call(
        _attn_kernel,
        grid=(S // BQ, H),
        in_specs=[
            pl.BlockSpec((1, BQ, HD), lambda i, h: (h, i, 0)),
            pl.BlockSpec((1, S, HD), lambda i, h: (h, 0, 0)),
            pl.BlockSpec((1, S, HD), lambda i, h: (h, 0, 0)),
            pl.BlockSpec((HD, C), lambda i, h: (h, 0)),
            pl.BlockSpec((1, C), lambda i, h: (0, 0)),
        ],
        out_specs=pl.BlockSpec((BQ, C), lambda i, h: (i, 0)),
        out_shape=jax.ShapeDtypeStruct((S, C), f32),
    )(q3, k3, v3, Wo.T, bo[None, :])

    return out
